# baseline (speedup 1.0000x reference)
import functools

import jax
import jax.numpy as jnp
from jax import lax
from jax.experimental import pallas as pl
from jax.experimental.pallas import tpu as pltpu
from jax.experimental.pallas import tpu_sc as plsc

_NUM_SAMPLES = 2048


def _make_sc_gather(total_elems: int):
    info = plsc.get_sparse_core_info()
    nc, ns = info.num_cores, info.num_subcores
    nw = nc * ns
    assert total_elems % nw == 0
    chunk = total_elems // nw

    mesh = plsc.VectorSubcoreMesh(core_axis_name="c", subcore_axis_name="s")

    @functools.partial(
        pl.kernel,
        out_type=(
            jax.ShapeDtypeStruct((total_elems,), jnp.float32),
            jax.ShapeDtypeStruct((total_elems,), jnp.float32),
        ),
        mesh=mesh,
        scratch_types=[
            pltpu.VMEM((chunk,), jnp.int32),
            pltpu.VMEM((chunk,), jnp.float32),
            pltpu.SemaphoreType.DMA,
        ],
    )
    def gather_kernel(c1_hbm, i1_hbm, c2_hbm, i2_hbm, o1_hbm, o2_hbm,
                      idx_v, vals_v, sem):
        wid = lax.axis_index("s") * nc + lax.axis_index("c")
        base = wid * chunk
        pltpu.sync_copy(i1_hbm.at[pl.ds(base, chunk)], idx_v)
        pltpu.async_copy(c1_hbm.at[idx_v], vals_v, sem).wait()
        pltpu.sync_copy(vals_v, o1_hbm.at[pl.ds(base, chunk)])
        pltpu.sync_copy(i2_hbm.at[pl.ds(base, chunk)], idx_v)
        pltpu.async_copy(c2_hbm.at[idx_v], vals_v, sem).wait()
        pltpu.sync_copy(vals_v, o2_hbm.at[pl.ds(base, chunk)])

    return gather_kernel


def _chamfer_body(s1_ref, s2t_ref, out_ref):
    a = s1_ref[0]
    bt = s2t_ref[0]
    sqa = jnp.sum(a * a, axis=1)
    sqb = jnp.sum(bt * bt, axis=0)
    cross = lax.dot_general(a, bt, (((1,), (0,)), ((), ())),
                            preferred_element_type=jnp.float32)
    d = sqa[:, None] + sqb[None, :] - 2.0 * cross
    rmin = jnp.min(d, axis=1)
    cmin = jnp.min(d, axis=0)
    out_ref[0, 0, 0] = jnp.mean(rmin) + jnp.mean(cmin)


def _chamfer_call(s1p, s2t):
    n, s, k = s1p.shape
    return pl.pallas_call(
        _chamfer_body,
        grid=(n,),
        in_specs=[
            pl.BlockSpec((1, s, k), lambda i: (i, 0, 0)),
            pl.BlockSpec((1, k, s), lambda i: (i, 0, 0)),
        ],
        out_specs=pl.BlockSpec((1, 1, 1), lambda i: (i, 0, 0),
                               memory_space=pltpu.SMEM),
        out_shape=jax.ShapeDtypeStruct((n, 1, 1), jnp.float32),
    )(s1p, s2t).reshape(n)


def kernel(cloud1, cloud2, num_samples):
    del num_samples
    n, p1, _ = cloud1.shape
    p2 = cloud2.shape[1]
    s = _NUM_SAMPLES

    key = jax.random.key(42)
    ka, kb = jax.random.split(key)
    idx1 = jax.random.randint(ka, (s,), 0, p1)
    idx2 = jax.random.randint(kb, (s,), 0, p2)

    batch_off = jnp.arange(n, dtype=jnp.int32)[:, None]
    idx1g = (batch_off * p1 + idx1[None, :].astype(jnp.int32)).reshape(-1)
    idx2g = (batch_off * p2 + idx2[None, :].astype(jnp.int32)).reshape(-1)
    coord = jnp.arange(3, dtype=jnp.int32)[None, :]
    idx1e = (idx1g[:, None] * 3 + coord).reshape(-1)
    idx2e = (idx2g[:, None] * 3 + coord).reshape(-1)

    gather = _make_sc_gather(n * s * 3)
    s1f, s2f = gather(cloud1.reshape(-1), idx1e,
                      cloud2.reshape(-1), idx2e)

    pad = ((0, 0), (0, 0), (0, 5))
    s1p = jnp.pad(s1f.reshape(n, s, 3), pad)
    s2p = jnp.pad(s2f.reshape(n, s, 3), pad)
    return _chamfer_call(s1p, s2p.transpose(0, 2, 1))
